# baseline (speedup 1.0000x reference)
import functools

import jax
import jax.numpy as jnp
from jax.experimental import pallas as pl


def _ln_relu(h, ls, lb):
    mu = jnp.mean(h, axis=-1, keepdims=True)
    var = jnp.mean((h - mu) ** 2, axis=-1, keepdims=True)
    h = (h - mu) * jax.lax.rsqrt(var + 1e-5) * ls + lb
    return jnp.maximum(h, 0.0)


def _fused_body(agent_ref, dyn_ref, stat_ref, act_ref,
                eW1_ref, eb1_ref, els_ref, elb_ref, eW2_ref, eb2_ref,
                nW1_ref, nb1_ref, nls_ref, nlb_ref, nW2_ref, nb2_ref,
                gW1_ref, gb1_ref, gls_ref, glb_ref, gW2_ref, gb2_ref,
                glob_out_ref, node_out_ref, *, bb, n):
    ndd = dyn_ref.shape[-1]
    nsd = stat_ref.shape[-1]
    na = ndd + nsd
    hd = eW1_ref.shape[-1]

    attr = jnp.concatenate(
        [dyn_ref[...].reshape(bb * n, ndd), stat_ref[...].reshape(bb * n, nsd)],
        axis=1)
    ctx = jnp.concatenate([agent_ref[...], act_ref[...]], axis=1)

    eW1 = eW1_ref[...]
    a_i = jnp.dot(attr, eW1[:na], preferred_element_type=jnp.float32)
    b_j = jnp.dot(attr, eW1[na:2 * na], preferred_element_type=jnp.float32)
    c_b = (jnp.dot(ctx, eW1[2 * na:], preferred_element_type=jnp.float32)
           + eb1_ref[...])

    a2 = a_i + jnp.broadcast_to(
        c_b.reshape(bb, 1, hd), (bb, n, hd)).reshape(bb * n, hd)

    ls = els_ref[...]
    ah = a2 - jnp.mean(a2, axis=-1, keepdims=True)
    bh = b_j - jnp.mean(b_j, axis=-1, keepdims=True)
    qa = jnp.sum(ah * ah, axis=-1, keepdims=True)
    qb = jnp.sum(bh * bh, axis=-1, keepdims=True)
    ones = jnp.ones((bb * n, 1), jnp.float32)
    lhs = jnp.concatenate([2.0 * bh, qb, ones], axis=1).reshape(bb, n, hd + 2)
    rhs = jnp.concatenate([ah, ones, qa], axis=1).reshape(bb, n, hd + 2)
    msq = jax.lax.dot_general(
        lhs, rhs, (((2,), (2,)), ((0,), (0,))),
        preferred_element_type=jnp.float32)
    inv3 = jax.lax.rsqrt(msq * (1.0 / hd) + 1e-5)
    inv4 = jnp.transpose(inv3, (1, 0, 2)).reshape(n, bb, n, 1)

    at = ah * ls
    bt = bh * ls
    b_t = jnp.transpose(bt.reshape(bb, n, hd), (1, 0, 2))
    b16 = b_t.astype(jnp.bfloat16).reshape(n, bb, 1, hd)
    a16 = at.astype(jnp.bfloat16).reshape(bb, n, hd)
    h = b16 + a16
    nlb = -elb_ref[...]
    r = jnp.maximum(h * inv4.astype(jnp.bfloat16), nlb.astype(jnp.bfloat16))
    inv_d = jax.lax.rsqrt(
        (qa + qb + 2.0 * jnp.sum(ah * bh, axis=-1, keepdims=True))
        * (1.0 / hd) + 1e-5)
    r_diag = jnp.maximum((at + bt) * inv_d, nlb)
    hsum = (jnp.sum(r, axis=0, dtype=jnp.float32)
            - (r_diag - (n - 1.0) * elb_ref[...]).reshape(bb, n, hd))
    gsum = jnp.sum(hsum, axis=1)

    eW2 = eW2_ref[...]
    agg = (jnp.dot(hsum.reshape(bb * n, hd) * (1.0 / (n - 1)), eW2,
                   preferred_element_type=jnp.float32) + eb2_ref[...])
    aggg = (jnp.dot(gsum * (1.0 / (n * (n - 1))), eW2,
                    preferred_element_type=jnp.float32) + eb2_ref[...])

    gc = ctx.shape[-1]
    ctx_rep = jnp.broadcast_to(ctx.reshape(bb, 1, gc), (bb, n, gc))
    x = jnp.concatenate([attr, ctx_rep.reshape(bb * n, gc), agg], axis=1)
    hn = _ln_relu(
        jnp.dot(x, nW1_ref[...], preferred_element_type=jnp.float32)
        + nb1_ref[...], nls_ref[...], nlb_ref[...])
    node_out = (jnp.dot(hn, nW2_ref[...], preferred_element_type=jnp.float32)
                + nb2_ref[...])
    node_out_ref[...] = node_out.reshape(bb, n, node_out.shape[-1])

    xg = jnp.concatenate([ctx, aggg], axis=1)
    hg = _ln_relu(
        jnp.dot(xg, gW1_ref[...], preferred_element_type=jnp.float32)
        + gb1_ref[...], gls_ref[...], glb_ref[...])
    glob_out_ref[...] = (
        jnp.dot(hg, gW2_ref[...], preferred_element_type=jnp.float32)
        + gb2_ref[...])


def kernel(agent_state, object_dyn_state, object_stat_state, action,
           edge_W1, edge_b1, edge_ls, edge_lb, edge_W2, edge_b2,
           node_W1, node_b1, node_ls, node_lb, node_W2, node_b2,
           glob_W1, glob_b1, glob_ls, glob_lb, glob_W2, glob_b2):
    b, gd = agent_state.shape
    _, n, ndd = object_dyn_state.shape
    nsd = object_stat_state.shape[-1]
    gcd = action.shape[-1]
    hd = edge_W1.shape[-1]

    bb = 64
    grid = (b // bb,)

    def row2d(d):
        return pl.BlockSpec((bb, d), lambda g: (g, 0))

    def row3d(d2, d3):
        return pl.BlockSpec((bb, d2, d3), lambda g: (g, 0, 0))

    def full(arr):
        return pl.BlockSpec(arr.shape, lambda g: (0,) * arr.ndim)

    r2 = lambda v: v.reshape(1, -1)
    weights = (edge_W1, r2(edge_b1), r2(edge_ls), r2(edge_lb), edge_W2,
               r2(edge_b2), node_W1, r2(node_b1), r2(node_ls), r2(node_lb),
               node_W2, r2(node_b2), glob_W1, r2(glob_b1), r2(glob_ls),
               r2(glob_lb), glob_W2, r2(glob_b2))

    glob_out, node_out = pl.pallas_call(
        functools.partial(_fused_body, bb=bb, n=n),
        grid=grid,
        in_specs=[row2d(gd), row3d(n, ndd), row3d(n, nsd), row2d(gcd)]
        + [full(w) for w in weights],
        out_specs=(pl.BlockSpec((bb, gd), lambda g: (g, 0)),
                   pl.BlockSpec((bb, n, ndd), lambda g: (g, 0, 0))),
        out_shape=(jax.ShapeDtypeStruct((b, gd), jnp.float32),
                   jax.ShapeDtypeStruct((b, n, ndd), jnp.float32)),
    )(agent_state, object_dyn_state, object_stat_state, action, *weights)

    return glob_out, node_out


# speedup vs baseline: 1.0348x; 1.0348x over previous
import functools

import jax
import jax.numpy as jnp
from jax.experimental import pallas as pl


def _ln_relu(h, ls, lb):
    mu = jnp.mean(h, axis=-1, keepdims=True)
    var = jnp.mean((h - mu) ** 2, axis=-1, keepdims=True)
    h = (h - mu) * jax.lax.rsqrt(var + 1e-5) * ls + lb
    return jnp.maximum(h, 0.0)


def _fused_body(agent_ref, dyn_ref, stat_ref, act_ref,
                eW1_ref, eb1_ref, els_ref, elb_ref, eW2_ref, eb2_ref,
                nW1_ref, nb1_ref, nls_ref, nlb_ref, nW2_ref, nb2_ref,
                gW1_ref, gb1_ref, gls_ref, glb_ref, gW2_ref, gb2_ref,
                glob_out_ref, node_out_ref, *, bb, n):
    ndd = dyn_ref.shape[-1]
    nsd = stat_ref.shape[-1]
    na = ndd + nsd
    hd = eW1_ref.shape[-1]
    row = lambda ref: ref[...].reshape(1, -1)
    eb1, els, elb, eb2 = row(eb1_ref), row(els_ref), row(elb_ref), row(eb2_ref)
    nb1, nls, nlb2, nb2 = row(nb1_ref), row(nls_ref), row(nlb_ref), row(nb2_ref)
    gb1, gls, glb, gb2 = row(gb1_ref), row(gls_ref), row(glb_ref), row(gb2_ref)

    attr = jnp.concatenate(
        [dyn_ref[...].reshape(bb * n, ndd), stat_ref[...].reshape(bb * n, nsd)],
        axis=1)
    ctx = jnp.concatenate([agent_ref[...], act_ref[...]], axis=1)

    eW1 = eW1_ref[...]
    a_i = jnp.dot(attr, eW1[:na], preferred_element_type=jnp.float32)
    b_j = jnp.dot(attr, eW1[na:2 * na], preferred_element_type=jnp.float32)
    c_b = (jnp.dot(ctx, eW1[2 * na:], preferred_element_type=jnp.float32)
           + eb1)

    a2 = a_i + jnp.broadcast_to(
        c_b.reshape(bb, 1, hd), (bb, n, hd)).reshape(bb * n, hd)

    ls = els
    ah = a2 - jnp.mean(a2, axis=-1, keepdims=True)
    bh = b_j - jnp.mean(b_j, axis=-1, keepdims=True)
    qa = jnp.sum(ah * ah, axis=-1, keepdims=True)
    qb = jnp.sum(bh * bh, axis=-1, keepdims=True)
    ones = jnp.ones((bb * n, 1), jnp.float32)
    lhs = jnp.concatenate([2.0 * bh, qb, ones], axis=1).reshape(bb, n, hd + 2)
    rhs = jnp.concatenate([ah, ones, qa], axis=1).reshape(bb, n, hd + 2)
    msq = jax.lax.dot_general(
        lhs, rhs, (((2,), (2,)), ((0,), (0,))),
        preferred_element_type=jnp.float32)
    inv3 = jax.lax.rsqrt(msq * (1.0 / hd) + 1e-5)
    inv4 = jnp.transpose(inv3, (1, 0, 2)).reshape(n, bb, n, 1)

    at = ah * ls
    bt = bh * ls
    b_t = jnp.transpose(bt.reshape(bb, n, hd), (1, 0, 2))
    h = b_t.reshape(n, bb, 1, hd) + at.reshape(bb, n, hd)
    nlb = -elb
    r = jnp.maximum(h * inv4, nlb)
    inv_d = jax.lax.rsqrt(
        (qa + qb + 2.0 * jnp.sum(ah * bh, axis=-1, keepdims=True))
        * (1.0 / hd) + 1e-5)
    r_diag = jnp.maximum((at + bt) * inv_d, nlb)
    hsum = (jnp.sum(r, axis=0, dtype=jnp.float32)
            - (r_diag - (n - 1.0) * elb).reshape(bb, n, hd))
    gsum = jnp.sum(hsum, axis=1)

    eW2 = eW2_ref[...]
    agg = (jnp.dot(hsum.reshape(bb * n, hd) * (1.0 / (n - 1)), eW2,
                   preferred_element_type=jnp.float32) + eb2)
    aggg = (jnp.dot(gsum * (1.0 / (n * (n - 1))), eW2,
                    preferred_element_type=jnp.float32) + eb2)

    gc = ctx.shape[-1]
    ctx_rep = jnp.broadcast_to(ctx.reshape(bb, 1, gc), (bb, n, gc))
    x = jnp.concatenate([attr, ctx_rep.reshape(bb * n, gc), agg], axis=1)
    hn = _ln_relu(
        jnp.dot(x, nW1_ref[...], preferred_element_type=jnp.float32)
        + nb1, nls, nlb2)
    node_out = (jnp.dot(hn, nW2_ref[...], preferred_element_type=jnp.float32)
                + nb2)
    node_out_ref[...] = node_out.reshape(bb, n, node_out.shape[-1])

    xg = jnp.concatenate([ctx, aggg], axis=1)
    hg = _ln_relu(
        jnp.dot(xg, gW1_ref[...], preferred_element_type=jnp.float32)
        + gb1, gls, glb)
    glob_out_ref[...] = (
        jnp.dot(hg, gW2_ref[...], preferred_element_type=jnp.float32)
        + gb2)


def kernel(agent_state, object_dyn_state, object_stat_state, action,
           edge_W1, edge_b1, edge_ls, edge_lb, edge_W2, edge_b2,
           node_W1, node_b1, node_ls, node_lb, node_W2, node_b2,
           glob_W1, glob_b1, glob_ls, glob_lb, glob_W2, glob_b2):
    b, gd = agent_state.shape
    _, n, ndd = object_dyn_state.shape
    nsd = object_stat_state.shape[-1]
    gcd = action.shape[-1]
    hd = edge_W1.shape[-1]

    bb = 64
    grid = (b // bb,)

    def row2d(d):
        return pl.BlockSpec((bb, d), lambda g: (g, 0))

    def row3d(d2, d3):
        return pl.BlockSpec((bb, d2, d3), lambda g: (g, 0, 0))

    def full(arr):
        return pl.BlockSpec(arr.shape, lambda g: (0,) * arr.ndim)

    weights = (edge_W1, edge_b1, edge_ls, edge_lb, edge_W2,
               edge_b2, node_W1, node_b1, node_ls, node_lb,
               node_W2, node_b2, glob_W1, glob_b1, glob_ls,
               glob_lb, glob_W2, glob_b2)

    glob_out, node_out = pl.pallas_call(
        functools.partial(_fused_body, bb=bb, n=n),
        grid=grid,
        in_specs=[row2d(gd), row3d(n, ndd), row3d(n, nsd), row2d(gcd)]
        + [full(w) for w in weights],
        out_specs=(pl.BlockSpec((bb, gd), lambda g: (g, 0)),
                   pl.BlockSpec((bb, n, ndd), lambda g: (g, 0, 0))),
        out_shape=(jax.ShapeDtypeStruct((b, gd), jnp.float32),
                   jax.ShapeDtypeStruct((b, n, ndd), jnp.float32)),
    )(agent_state, object_dyn_state, object_stat_state, action, *weights)

    return glob_out, node_out


# speedup vs baseline: 1.4404x; 1.3920x over previous
import functools

import jax
import jax.numpy as jnp
from jax.experimental import pallas as pl


def _ln_relu(h, ls, lb):
    mu = jnp.mean(h, axis=-1, keepdims=True)
    var = jnp.mean((h - mu) ** 2, axis=-1, keepdims=True)
    h = (h - mu) * jax.lax.rsqrt(var + 1e-5) * ls + lb
    return jnp.maximum(h, 0.0)


def _fused_body(agent_ref, dyn_ref, stat_ref, act_ref,
                eW1_ref, eb1_ref, els_ref, elb_ref, eW2_ref, eb2_ref,
                nW1_ref, nb1_ref, nls_ref, nlb_ref, nW2_ref, nb2_ref,
                gW1_ref, gb1_ref, gls_ref, glb_ref, gW2_ref, gb2_ref,
                glob_out_ref, node_out_ref, *, bb, n, jc):
    ndd = dyn_ref.shape[1]
    nsd = stat_ref.shape[1]
    na = ndd + nsd
    hd = eW1_ref.shape[-1]
    row = lambda ref: ref[...].reshape(1, -1)
    eb1, els, elb, eb2 = row(eb1_ref), row(els_ref), row(elb_ref), row(eb2_ref)
    nb1, nls, nlb2, nb2 = row(nb1_ref), row(nls_ref), row(nlb_ref), row(nb2_ref)
    gb1, gls, glb, gb2 = row(gb1_ref), row(gls_ref), row(glb_ref), row(gb2_ref)

    dyn_nb = jnp.transpose(dyn_ref[...], (0, 2, 1))
    stat_nb = jnp.transpose(stat_ref[...], (0, 2, 1))
    attr_nb3 = jnp.concatenate([dyn_nb, stat_nb], axis=-1)
    attr_nb = attr_nb3.reshape(n * bb, na)
    attr3 = jnp.transpose(attr_nb3, (1, 0, 2))
    attr = attr3.reshape(bb * n, na)
    ctx = jnp.concatenate(
        [jnp.transpose(agent_ref[...], (1, 0)),
         jnp.transpose(act_ref[...], (1, 0))], axis=1)

    eW1 = eW1_ref[...]
    a_i = jnp.dot(attr, eW1[:na], preferred_element_type=jnp.float32)
    bt_nb = jnp.dot(attr_nb, eW1[na:2 * na],
                    preferred_element_type=jnp.float32)
    b_j = jnp.dot(attr, eW1[na:2 * na], preferred_element_type=jnp.float32)
    c_b = (jnp.dot(ctx, eW1[2 * na:], preferred_element_type=jnp.float32)
           + eb1)

    a2 = a_i + jnp.broadcast_to(
        c_b.reshape(bb, 1, hd), (bb, n, hd)).reshape(bb * n, hd)

    ls = els
    ah = a2 - jnp.mean(a2, axis=-1, keepdims=True)
    bh = b_j - jnp.mean(b_j, axis=-1, keepdims=True)
    bh_nb = bt_nb - jnp.mean(bt_nb, axis=-1, keepdims=True)
    qa = jnp.sum(ah * ah, axis=-1, keepdims=True)
    qb = jnp.sum(bh * bh, axis=-1, keepdims=True)
    ones = jnp.ones((bb * n, 1), jnp.float32)
    lhs = jnp.concatenate([2.0 * bh, qb, ones], axis=1).reshape(bb, n, hd + 2)
    rhs = jnp.concatenate([ah, ones, qa], axis=1).reshape(bb, n, hd + 2)
    msq = jax.lax.dot_general(
        lhs, rhs, (((2,), (2,)), ((0,), (0,))),
        preferred_element_type=jnp.float32)
    inv3 = jax.lax.rsqrt(msq * (1.0 / hd) + 1e-5)

    at3 = (ah * ls).reshape(bb, n, hd)
    bt = bh * ls
    b_t = (bh_nb * ls).reshape(n, bb, hd)
    nlb = -elb
    acc = None
    for c in range(0, n, jc):
        inv_c = jnp.transpose(inv3[:, c:c + jc, :],
                              (1, 0, 2)).reshape(jc, bb, n, 1)
        hc = b_t[c:c + jc].reshape(jc, bb, 1, hd) + at3
        rc = jnp.maximum(hc * inv_c, nlb)
        s = jnp.sum(rc, axis=0, dtype=jnp.float32)
        acc = s if acc is None else acc + s
    inv_d = jax.lax.rsqrt(
        (qa + qb + 2.0 * jnp.sum(ah * bh, axis=-1, keepdims=True))
        * (1.0 / hd) + 1e-5)
    r_diag = jnp.maximum((ah * ls + bt) * inv_d, nlb)
    hsum = acc - (r_diag - (n - 1.0) * elb).reshape(bb, n, hd)
    gsum = jnp.sum(hsum, axis=1)

    eW2 = eW2_ref[...]
    agg = (jnp.dot(hsum.reshape(bb * n, hd) * (1.0 / (n - 1)), eW2,
                   preferred_element_type=jnp.float32) + eb2)
    aggg = (jnp.dot(gsum * (1.0 / (n * (n - 1))), eW2,
                    preferred_element_type=jnp.float32) + eb2)

    gc = ctx.shape[-1]
    ctx_rep = jnp.broadcast_to(ctx.reshape(bb, 1, gc), (bb, n, gc))
    x = jnp.concatenate([attr, ctx_rep.reshape(bb * n, gc), agg], axis=1)
    hn = _ln_relu(
        jnp.dot(x, nW1_ref[...], preferred_element_type=jnp.float32)
        + nb1, nls, nlb2)
    node_out = jax.lax.dot_general(
        hn, nW2_ref[...], (((1,), (1,)), ((), ())),
        preferred_element_type=jnp.float32) + nb2
    node_out_ref[...] = jnp.transpose(
        jnp.transpose(node_out.reshape(bb, n, ndd), (1, 0, 2)),
        (0, 2, 1))

    xg = jnp.concatenate([ctx, aggg], axis=1)
    hg = _ln_relu(
        jnp.dot(xg, gW1_ref[...], preferred_element_type=jnp.float32)
        + gb1, gls, glb)
    glob = jax.lax.dot_general(
        hg, gW2_ref[...], (((1,), (1,)), ((), ())),
        preferred_element_type=jnp.float32) + gb2
    glob_out_ref[...] = jnp.transpose(glob, (1, 0))


def kernel(agent_state, object_dyn_state, object_stat_state, action,
           edge_W1, edge_b1, edge_ls, edge_lb, edge_W2, edge_b2,
           node_W1, node_b1, node_ls, node_lb, node_W2, node_b2,
           glob_W1, glob_b1, glob_ls, glob_lb, glob_W2, glob_b2):
    b, gd = agent_state.shape
    _, n, ndd = object_dyn_state.shape
    nsd = object_stat_state.shape[-1]
    gcd = action.shape[-1]
    hd = edge_W1.shape[-1]

    bb = 128
    jc = 4
    grid = (b // bb,)

    ag_t = jnp.transpose(agent_state, (1, 0))
    ac_t = jnp.transpose(action, (1, 0))
    dyn_t = jnp.transpose(object_dyn_state, (1, 2, 0))
    stat_t = jnp.transpose(object_stat_state, (1, 2, 0))
    nW2_t = jnp.transpose(node_W2, (1, 0))
    gW2_t = jnp.transpose(glob_W2, (1, 0))

    def full(arr):
        return pl.BlockSpec(arr.shape, lambda g: (0,) * arr.ndim)

    weights = (edge_W1, edge_b1, edge_ls, edge_lb, edge_W2,
               edge_b2, node_W1, node_b1, node_ls, node_lb,
               nW2_t, node_b2, glob_W1, glob_b1, glob_ls,
               glob_lb, gW2_t, glob_b2)

    glob_out_t, node_out_t = pl.pallas_call(
        functools.partial(_fused_body, bb=bb, n=n, jc=jc),
        grid=grid,
        in_specs=[pl.BlockSpec((gd, bb), lambda g: (0, g)),
                  pl.BlockSpec((n, ndd, bb), lambda g: (0, 0, g)),
                  pl.BlockSpec((n, nsd, bb), lambda g: (0, 0, g)),
                  pl.BlockSpec((gcd, bb), lambda g: (0, g))]
        + [full(w) for w in weights],
        out_specs=(pl.BlockSpec((gd, bb), lambda g: (0, g)),
                   pl.BlockSpec((n, ndd, bb), lambda g: (0, 0, g))),
        out_shape=(jax.ShapeDtypeStruct((gd, b), jnp.float32),
                   jax.ShapeDtypeStruct((n, ndd, b), jnp.float32)),
    )(ag_t, dyn_t, stat_t, ac_t, *weights)

    return (jnp.transpose(glob_out_t, (1, 0)),
            jnp.transpose(node_out_t, (2, 0, 1)))
